# Initial kernel scaffold; baseline (speedup 1.0000x reference)
import jax
import jax.numpy as jnp
from jax.experimental import pallas as pl


def kernel(h, edge_index, bias):
    raise NotImplementedError("write your pallas kernel here")


# speedup vs baseline: 8.9876x; 8.9876x over previous
import functools

import jax
import jax.numpy as jnp
from jax import lax
from jax.experimental import pallas as pl
from jax.experimental.pallas import tpu as pltpu
from jax.experimental.pallas import tpu_sc as plsc

_C = 1.0

_NC = 2
_NS = 16
_NW = _NC * _NS
_L = 16


def _log0_body(h_ref, out_ref, n2_ref):
    x = h_ref[...]
    sqrt_c = _C ** 0.5
    alpha = jnp.maximum(sqrt_c * x[:, 0:1], 1.0 + 1e-7)
    s = jnp.sqrt(alpha * alpha - 1.0)
    coef = jnp.log(alpha + s) / s
    out = coef * x
    col = lax.broadcasted_iota(jnp.int32, out.shape, 1)
    out = jnp.where(col == 0, 0.0, out)
    out_ref[...] = out
    n2_ref[...] = 0.5 * jnp.sum(out * out, axis=1)


def _log0(h):
    n, d = h.shape
    return pl.pallas_call(
        _log0_body,
        grid=(1,),
        in_specs=[pl.BlockSpec((n, d), lambda i: (0, 0))],
        out_specs=[pl.BlockSpec((n, d), lambda i: (0, 0)),
                   pl.BlockSpec((n,), lambda i: (0,))],
        out_shape=[jax.ShapeDtypeStruct((n, d), jnp.float32),
                   jax.ShapeDtypeStruct((n,), jnp.float32)],
    )(h)


def _make_sc_dot(n, e, d, ew, b):
    nb = ew // b
    assert nb % 3 == 2
    ngr = b // _L
    nq = d // _L
    mesh = plsc.VectorSubcoreMesh(core_axis_name="c", subcore_axis_name="s")

    @functools.partial(
        pl.kernel,
        mesh=mesh,
        out_type=jax.ShapeDtypeStruct((e,), jnp.float32),
        compiler_params=pltpu.CompilerParams(needs_layout_passes=False),
        scratch_types=[
            pltpu.VMEM((ew,), jnp.int32),
            pltpu.VMEM((ew,), jnp.int32),
            pltpu.VMEM((n,), jnp.float32),
            pltpu.VMEM((b, d), jnp.float32),
            pltpu.VMEM((b, d), jnp.float32),
            pltpu.VMEM((b, d), jnp.float32),
            pltpu.VMEM((ew,), jnp.float32),
            pltpu.SemaphoreType.DMA,
            pltpu.SemaphoreType.DMA,
            pltpu.SemaphoreType.DMA,
            pltpu.SemaphoreType.DMA,
            pltpu.SemaphoreType.DMA,
            pltpu.SemaphoreType.DMA,
        ],
    )
    def sc_dot(tan_hbm, n2_hbm, eidx_hbm, out_hbm, idx_row, idx_col, n2_v,
               c0, c1, c2, out_all, si0, si1, si2, sj0, sj1, sj2):
        bufs = (c0, c1, c2)
        sems_i = (si0, si1, si2)
        sems_j = (sj0, sj1, sj2)
        wid = lax.axis_index("s") * _NC + lax.axis_index("c")
        base_w = wid * ew
        pltpu.sync_copy(eidx_hbm.at[pl.ds(base_w, ew)], idx_row)
        pltpu.sync_copy(eidx_hbm.at[pl.ds(e + base_w, ew)], idx_col)
        pltpu.sync_copy(n2_hbm, n2_v)

        def fire_i(m, k):
            pltpu.async_copy(
                tan_hbm.at[idx_row.at[pl.ds(m * b, b)]], bufs[k], sems_i[k])

        def fire_jadd(m, k):
            pltpu.async_copy(
                tan_hbm.at[idx_col.at[pl.ds(m * b, b)]], bufs[k], sems_j[k],
                add=True)

        def drain(k, sems):
            pltpu.make_async_copy(
                tan_hbm.at[pl.ds(0, b)], bufs[k], sems[k]).wait()

        lanes = lax.iota(jnp.int32, _L)

        def compute(bi, k):
            buf = bufs[k]
            for g in range(ngr):
                def estep(l, acc):
                    e2 = g * _L + l
                    facc = None
                    for q in range(nq):
                        v = buf[e2, pl.ds(q * _L, _L)]
                        facc = v * v if facc is None else facc + v * v
                    return jnp.where(lanes == l, jnp.sum(facc), acc)

                acc = lax.fori_loop(0, _L, estep,
                                    jnp.zeros((_L,), jnp.float32), unroll=4)
                ids_i = idx_row[pl.ds(bi * b + g * _L, _L)]
                ids_j = idx_col[pl.ds(bi * b + g * _L, _L)]
                out_all[pl.ds(bi * b + g * _L, _L)] = (
                    0.5 * acc
                    - plsc.load_gather(n2_v, [ids_i])
                    - plsc.load_gather(n2_v, [ids_j]))

        def sub(m, k):
            k1 = (k + 1) % 3
            k2 = (k + 2) % 3
            fire_i(m + 2, k2)
            drain(k1, sems_i)
            fire_jadd(m + 1, k1)
            drain(k, sems_j)
            compute(m, k)

        fire_i(0, 0)
        drain(0, sems_i)
        fire_jadd(0, 0)
        fire_i(1, 1)

        def body(j, carry):
            m = 3 * j
            sub(m, 0)
            sub(m + 1, 1)
            sub(m + 2, 2)
            return carry

        lax.fori_loop(0, (nb - 2) // 3, body, 0)
        drain(1, sems_i)
        fire_jadd(nb - 1, 1)
        drain(0, sems_j)
        compute(nb - 2, 0)
        drain(1, sems_j)
        compute(nb - 1, 1)
        pltpu.sync_copy(out_all, out_hbm.at[pl.ds(base_w, ew)])

    return sc_dot


def kernel(h, edge_index, bias):
    n, d = h.shape
    e = edge_index.shape[1]
    assert e % _NW == 0 and d % _L == 0
    ew = e // _NW
    b = 80
    assert ew % b == 0 and b % 8 == 0 and ew % 8 == 0
    h_tan, n2h = _log0(h)
    scores = _make_sc_dot(n, e, d, ew, b)(h_tan, n2h, jnp.ravel(edge_index))
    return scores + bias


# speedup vs baseline: 10.1689x; 1.1314x over previous
import functools

import jax
import jax.numpy as jnp
from jax import lax
from jax.experimental import pallas as pl
from jax.experimental.pallas import tpu as pltpu
from jax.experimental.pallas import tpu_sc as plsc

_C = 1.0

_NC = 2
_NS = 16
_NW = _NC * _NS
_L = 16


def _log0_body(h_ref, out_ref):
    x = h_ref[...]
    sqrt_c = _C ** 0.5
    alpha = jnp.maximum(sqrt_c * x[:, 0:1], 1.0 + 1e-7)
    s = jnp.sqrt(alpha * alpha - 1.0)
    coef = jnp.log(alpha + s) / s
    out = coef * x
    col = lax.broadcasted_iota(jnp.int32, out.shape, 1)
    out = jnp.where(col == 0, 0.0, out)
    n2h = 0.5 * jnp.sum(out * out, axis=1, keepdims=True)
    out_ref[...] = jnp.where(col == 0, n2h, out)


def _log0(h):
    n, d = h.shape
    blk = 1000
    return pl.pallas_call(
        _log0_body,
        grid=(n // blk,),
        in_specs=[pl.BlockSpec((blk, d), lambda i: (i, 0))],
        out_specs=pl.BlockSpec((blk, d), lambda i: (i, 0)),
        out_shape=jax.ShapeDtypeStruct((n, d), jnp.float32),
    )(h)


def _make_sc_dot(n, e, d, ew, b):
    nb = ew // b
    assert nb % 3 == 2
    ngr = b // _L
    nq = d // _L
    mesh = plsc.VectorSubcoreMesh(core_axis_name="c", subcore_axis_name="s")

    @functools.partial(
        pl.kernel,
        mesh=mesh,
        out_type=jax.ShapeDtypeStruct((e,), jnp.float32),
        compiler_params=pltpu.CompilerParams(needs_layout_passes=False),
        scratch_types=[
            pltpu.VMEM((ew,), jnp.int32),
            pltpu.VMEM((b,), jnp.int32),
            pltpu.VMEM((b,), jnp.int32),
            pltpu.VMEM((b,), jnp.int32),
            pltpu.VMEM((b, d), jnp.float32),
            pltpu.VMEM((b, d), jnp.float32),
            pltpu.VMEM((b, d), jnp.float32),
            pltpu.VMEM((b,), jnp.float32),
            pltpu.VMEM((b,), jnp.float32),
            pltpu.VMEM((b,), jnp.float32),
            pltpu.VMEM_SHARED((n, d), jnp.float32),
            pltpu.SemaphoreType.DMA,
            pltpu.SemaphoreType.DMA,
            pltpu.SemaphoreType.DMA,
            pltpu.SemaphoreType.DMA,
            pltpu.SemaphoreType.DMA,
            pltpu.SemaphoreType.DMA,
            pltpu.SemaphoreType.DMA,
            pltpu.SemaphoreType.DMA,
            pltpu.SemaphoreType.DMA,
            pltpu.SemaphoreType.DMA,
            pltpu.SemaphoreType.DMA,
            pltpu.SemaphoreType.DMA,
        ],
    )
    def sc_dot(tan_hbm, eidx_hbm, out_hbm, idx_row, q0, q1, q2,
               c0, c1, c2, o0, o1, o2, tab_sh,
               si0, si1, si2, sj0, sj1, sj2, so0, so1, so2,
               sq0, sq1, sq2):
        bufs = (c0, c1, c2)
        obufs = (o0, o1, o2)
        qbufs = (q0, q1, q2)
        sems_i = (si0, si1, si2)
        sems_j = (sj0, sj1, sj2)
        sems_o = (so0, so1, so2)
        sems_q = (sq0, sq1, sq2)
        wid = lax.axis_index("s") * _NC + lax.axis_index("c")
        base_w = wid * ew
        @pl.when(lax.axis_index("s") == 0)
        def _stage_table():
            pltpu.sync_copy(tan_hbm, tab_sh)

        pltpu.sync_copy(eidx_hbm.at[pl.ds(base_w, ew)], idx_row)
        plsc.subcore_barrier()

        def fire_i(m, k):
            pltpu.async_copy(
                tan_hbm.at[idx_row.at[pl.ds(m * b, b)]], bufs[k], sems_i[k])
            pltpu.async_copy(
                eidx_hbm.at[pl.ds(e + base_w + m * b, b)], qbufs[k],
                sems_q[k])

        def fire_jadd(m, k):
            pltpu.async_copy(
                tab_sh.at[qbufs[k]], bufs[k], sems_j[k], add=True)

        def drain(k, sems):
            pltpu.make_async_copy(
                tan_hbm.at[pl.ds(0, b)], bufs[k], sems[k]).wait()

        def drain_q(k):
            pltpu.make_async_copy(
                eidx_hbm.at[pl.ds(0, b)], qbufs[k], sems_q[k]).wait()

        lanes = lax.iota(jnp.int32, _L)

        def compute(bi, k):
            buf = bufs[k]
            ob = obufs[k]

            @pl.when(bi >= 3)
            def _reclaim_out():
                pltpu.make_async_copy(
                    ob, out_hbm.at[pl.ds(base_w, b)], sems_o[k]).wait()

            for g in range(ngr):
                def estep(l, acc):
                    e2 = g * _L + l
                    v0 = buf[e2, pl.ds(0, _L)]
                    facc = jnp.where(lanes == 0, -2.0 * v0, v0 * v0)
                    for q in range(1, nq):
                        v = buf[e2, pl.ds(q * _L, _L)]
                        facc = facc + v * v
                    return jnp.where(lanes == l, jnp.sum(facc), acc)

                acc = lax.fori_loop(0, _L, estep,
                                    jnp.zeros((_L,), jnp.float32), unroll=4)
                ob[pl.ds(g * _L, _L)] = 0.5 * acc
            pltpu.async_copy(
                ob, out_hbm.at[pl.ds(base_w + bi * b, b)], sems_o[k])

        def sub(m, k):
            k1 = (k + 1) % 3
            k2 = (k + 2) % 3
            fire_i(m + 2, k2)
            drain(k1, sems_i)
            drain_q(k1)
            fire_jadd(m + 1, k1)
            drain(k, sems_j)
            compute(m, k)

        fire_i(0, 0)
        drain(0, sems_i)
        drain_q(0)
        fire_jadd(0, 0)
        fire_i(1, 1)

        def body(j, carry):
            m = 3 * j
            sub(m, 0)
            sub(m + 1, 1)
            sub(m + 2, 2)
            return carry

        lax.fori_loop(0, (nb - 2) // 3, body, 0)
        drain(1, sems_i)
        drain_q(1)
        fire_jadd(nb - 1, 1)
        drain(0, sems_j)
        compute(nb - 2, 0)
        drain(1, sems_j)
        compute(nb - 1, 1)
        for k in (2, 0, 1):
            pltpu.make_async_copy(
                obufs[k], out_hbm.at[pl.ds(base_w, b)], sems_o[k]).wait()

    return sc_dot


def kernel(h, edge_index, bias):
    n, d = h.shape
    e = edge_index.shape[1]
    assert e % _NW == 0 and d % _L == 0
    ew = e // _NW
    b = 80
    assert ew % b == 0 and b % 8 == 0 and ew % 8 == 0
    tab = _log0(h)
    scores = _make_sc_dot(n, e, d, ew, b)(tab, jnp.ravel(edge_index))
    return scores + bias


# speedup vs baseline: 11.3729x; 1.1184x over previous
import functools

import jax
import jax.numpy as jnp
from jax import lax
from jax.experimental import pallas as pl
from jax.experimental.pallas import tpu as pltpu
from jax.experimental.pallas import tpu_sc as plsc

_C = 1.0

_NC = 2
_NS = 16
_NW = _NC * _NS
_L = 16


def _log0_body(h_ref, out_ref):
    x = h_ref[...]
    sqrt_c = _C ** 0.5
    alpha = jnp.maximum(sqrt_c * x[:, 0:1], 1.0 + 1e-7)
    s = jnp.sqrt(alpha * alpha - 1.0)
    coef = jnp.log(alpha + s) / s
    out = coef * x
    col = lax.broadcasted_iota(jnp.int32, out.shape, 1)
    out = jnp.where(col == 0, 0.0, out)
    n2h = 0.5 * jnp.sum(out * out, axis=1, keepdims=True)
    out_ref[...] = jnp.where(col == 0, n2h, out)


def _log0(h):
    n, d = h.shape
    blk = 1000
    return pl.pallas_call(
        _log0_body,
        grid=(n // blk,),
        in_specs=[pl.BlockSpec((blk, d), lambda i: (i, 0))],
        out_specs=pl.BlockSpec((blk, d), lambda i: (i, 0)),
        out_shape=jax.ShapeDtypeStruct((n, d), jnp.float32),
    )(h)


def _make_sc_dot(n, e, d, ew, b):
    nb = ew // b
    assert nb % 3 == 2
    ngr = b // _L
    nq = d // _L
    mesh = plsc.VectorSubcoreMesh(core_axis_name="c", subcore_axis_name="s")

    @functools.partial(
        pl.kernel,
        mesh=mesh,
        out_type=jax.ShapeDtypeStruct((e,), jnp.float32),
        compiler_params=pltpu.CompilerParams(needs_layout_passes=False),
        scratch_types=[
            pltpu.VMEM((ew,), jnp.int32),
            pltpu.VMEM((b,), jnp.int32),
            pltpu.VMEM((b,), jnp.int32),
            pltpu.VMEM((b,), jnp.int32),
            pltpu.VMEM((b, d), jnp.float32),
            pltpu.VMEM((b, d), jnp.float32),
            pltpu.VMEM((b, d), jnp.float32),
            pltpu.VMEM((b,), jnp.float32),
            pltpu.VMEM((b,), jnp.float32),
            pltpu.VMEM((b,), jnp.float32),
            pltpu.VMEM_SHARED((n, d), jnp.float32),
            pltpu.SemaphoreType.DMA,
            pltpu.SemaphoreType.DMA,
            pltpu.SemaphoreType.DMA,
            pltpu.SemaphoreType.DMA,
            pltpu.SemaphoreType.DMA,
            pltpu.SemaphoreType.DMA,
            pltpu.SemaphoreType.DMA,
            pltpu.SemaphoreType.DMA,
            pltpu.SemaphoreType.DMA,
            pltpu.SemaphoreType.DMA,
            pltpu.SemaphoreType.DMA,
            pltpu.SemaphoreType.DMA,
        ],
    )
    def sc_dot(tan_hbm, eidx_hbm, out_hbm, idx_row, q0, q1, q2,
               c0, c1, c2, o0, o1, o2, tab_sh,
               si0, si1, si2, sj0, sj1, sj2, so0, so1, so2,
               sq0, sq1, sq2):
        bufs = (c0, c1, c2)
        obufs = (o0, o1, o2)
        qbufs = (q0, q1, q2)
        sems_i = (si0, si1, si2)
        sems_j = (sj0, sj1, sj2)
        sems_o = (so0, so1, so2)
        sems_q = (sq0, sq1, sq2)
        wid = lax.axis_index("s") * _NC + lax.axis_index("c")
        base_w = wid * ew
        @pl.when(lax.axis_index("s") == 0)
        def _stage_table():
            pltpu.sync_copy(tan_hbm, tab_sh)

        pltpu.sync_copy(eidx_hbm.at[pl.ds(base_w, ew)], idx_row)
        plsc.subcore_barrier()

        def fire_i(m, k):
            pltpu.async_copy(
                tan_hbm.at[idx_row.at[pl.ds(m * b, b)]], bufs[k], sems_i[k])
            pltpu.async_copy(
                eidx_hbm.at[pl.ds(e + base_w + m * b, b)], qbufs[k],
                sems_q[k])

        def fire_jadd(m, k):
            tab = tab_sh if k != 2 else tan_hbm
            pltpu.async_copy(tab.at[qbufs[k]], bufs[k], sems_j[k], add=True)

        def drain(k, sems):
            pltpu.make_async_copy(
                tan_hbm.at[pl.ds(0, b)], bufs[k], sems[k]).wait()

        def drain_q(k):
            pltpu.make_async_copy(
                eidx_hbm.at[pl.ds(0, b)], qbufs[k], sems_q[k]).wait()

        lanes = lax.iota(jnp.int32, _L)

        def compute(bi, k):
            buf = bufs[k]
            ob = obufs[k]

            @pl.when(bi >= 3)
            def _reclaim_out():
                pltpu.make_async_copy(
                    ob, out_hbm.at[pl.ds(base_w, b)], sems_o[k]).wait()

            for g in range(ngr):
                def estep(l, acc):
                    e2 = g * _L + l
                    v0 = buf[e2, pl.ds(0, _L)]
                    facc = jnp.where(lanes == 0, -2.0 * v0, v0 * v0)
                    for q in range(1, nq):
                        v = buf[e2, pl.ds(q * _L, _L)]
                        facc = facc + v * v
                    return jnp.where(lanes == l, jnp.sum(facc), acc)

                acc = lax.fori_loop(0, _L, estep,
                                    jnp.zeros((_L,), jnp.float32), unroll=8)
                ob[pl.ds(g * _L, _L)] = 0.5 * acc
            pltpu.async_copy(
                ob, out_hbm.at[pl.ds(base_w + bi * b, b)], sems_o[k])

        def sub(m, k):
            k1 = (k + 1) % 3
            k2 = (k + 2) % 3
            fire_i(m + 2, k2)
            drain(k1, sems_i)
            drain_q(k1)
            fire_jadd(m + 1, k1)
            drain(k, sems_j)
            compute(m, k)

        fire_i(0, 0)
        drain(0, sems_i)
        drain_q(0)
        fire_jadd(0, 0)
        fire_i(1, 1)

        def body(j, carry):
            m = 3 * j
            sub(m, 0)
            sub(m + 1, 1)
            sub(m + 2, 2)
            return carry

        lax.fori_loop(0, (nb - 2) // 3, body, 0)
        drain(1, sems_i)
        drain_q(1)
        fire_jadd(nb - 1, 1)
        drain(0, sems_j)
        compute(nb - 2, 0)
        drain(1, sems_j)
        compute(nb - 1, 1)
        for k in (2, 0, 1):
            pltpu.make_async_copy(
                obufs[k], out_hbm.at[pl.ds(base_w, b)], sems_o[k]).wait()

    return sc_dot


def kernel(h, edge_index, bias):
    n, d = h.shape
    e = edge_index.shape[1]
    assert e % _NW == 0 and d % _L == 0
    ew = e // _NW
    b = 80
    assert ew % b == 0 and b % 8 == 0 and ew % 8 == 0
    tab = _log0(h)
    scores = _make_sc_dot(n, e, d, ew, b)(tab, jnp.ravel(edge_index))
    return scores + bias
